# baseline (speedup 1.0000x reference)
import jax
import jax.numpy as jnp
from jax import lax
from jax.experimental import pallas as pl
from jax.experimental.pallas import tpu as pltpu
from jax.experimental.pallas import tpu_sc as plsc

VOCAB_SIZES = [99991, 100003, 100019, 100043, 100049, 100057, 100069, 100103]
H = len(VOCAB_SIZES)
HP = 128
D = 64
B, S = 4, 2048

_off = []
_acc = 0
for _v in VOCAB_SIZES:
    _off.append(_acc)
    _acc += _v

NC, NS, L = 2, 16, 16
NW = NC * NS
SW = S * B // NW
SC_CHUNK = 16
NCHUNK = SW // SC_CHUNK

NBUF = 4
DEPTH = 2


def _body(ids_hbm, table_hbm, off_hbm, out_hbm, stag_v, zs_v, bufs_v, *sems):
    gsems = sems[:NBUF]
    wsems = sems[NBUF:]
    wid = lax.axis_index("s") * NC + lax.axis_index("c")
    b = wid // (NW // B)
    s0 = (wid % (NW // B)) * SW

    pltpu.sync_copy(ids_hbm.at[b, pl.ds(s0, SW)], stag_v)

    iota = lax.iota(jnp.int32, L)
    zero = iota - iota

    def zfill(k, _):
        zs_v[pl.ds(k * L, L)] = zero
        return 0

    lax.fori_loop(0, SW // L, zfill, 0)

    pltpu.async_copy(off_hbm.at[zs_v], stag_v, gsems[0], add=True).wait()

    w = [None] * NCHUNK

    def fire(j):
        bi = j % NBUF
        for r in range(SC_CHUNK):
            pltpu.async_copy(
                table_hbm.at[stag_v.at[j * SC_CHUNK + r, pl.ds(0, H)]],
                bufs_v.at[bi, r],
                gsems[bi],
            )

    def retire(j):
        bi = j % NBUF
        pltpu.make_async_copy(
            out_hbm.at[b, pl.ds(0, SC_CHUNK)],
            bufs_v.at[bi],
            gsems[bi],
        ).wait()
        w[j] = pltpu.async_copy(
            bufs_v.at[bi],
            out_hbm.at[b, pl.ds(s0 + j * SC_CHUNK, SC_CHUNK)],
            wsems[bi],
        )

    for j in range(NCHUNK):
        if j >= NBUF:
            w[j - NBUF].wait()
        fire(j)
        if j >= DEPTH:
            retire(j - DEPTH)
    for j in range(NCHUNK - DEPTH, NCHUNK):
        retire(j)
    for j in range(NCHUNK - NBUF, NCHUNK):
        w[j].wait()


@jax.jit
def kernel(input_ids, table):
    ids_p = jnp.pad(input_ids, ((0, 0), (0, 0), (0, HP - H)))
    offs = jnp.asarray([_off + [0] * (HP - H)], dtype=jnp.int32)
    mesh = plsc.VectorSubcoreMesh(core_axis_name="c", subcore_axis_name="s")
    out = pl.kernel(
        _body,
        mesh=mesh,
        out_type=jax.ShapeDtypeStruct((B, S, H, D), jnp.float32),
        compiler_params=pltpu.CompilerParams(use_tc_tiling_on_sc=False),
        scratch_types=[
            pltpu.VMEM((SW, HP), jnp.int32),
            pltpu.VMEM((SW,), jnp.int32),
            pltpu.VMEM((NBUF, SC_CHUNK, H, D), jnp.float32),
        ]
        + [pltpu.SemaphoreType.DMA] * (2 * NBUF),
    )(ids_p, table, offs)
    return out


# speedup vs baseline: 1.5592x; 1.5592x over previous
import jax
import jax.numpy as jnp
from jax import lax
from jax.experimental import pallas as pl
from jax.experimental.pallas import tpu as pltpu
from jax.experimental.pallas import tpu_sc as plsc

VOCAB_SIZES = [99991, 100003, 100019, 100043, 100049, 100057, 100069, 100103]
H = len(VOCAB_SIZES)
D = 64
B, S = 4, 2048
N = B * S * H

_off = []
_acc = 0
for _v in VOCAB_SIZES:
    _off.append(_acc)
    _acc += _v
OFF16 = tuple(_off * 2)

NC, NS, L = 2, 16, 16
NW = NC * NS
PER_W = N // NW
CHUNK = 128
NCHUNK = PER_W // CHUNK

NBUF = 4
DEPTH = 2


def _body(ids_hbm, table_hbm, off_hbm, out_hbm, stag_v, bufs_v, off_v, *sems):
    gsems = sems[:NBUF]
    wsems = sems[NBUF:]
    wid = lax.axis_index("s") * NC + lax.axis_index("c")
    base = wid * PER_W
    ob = wid // (NW // B)
    or0 = (wid % (NW // B)) * PER_W

    pltpu.sync_copy(ids_hbm.at[pl.ds(base, PER_W)], stag_v)

    pltpu.sync_copy(off_hbm, off_v)
    off = off_v[...]

    def add_step(k, _):
        sl = pl.ds(k * L, L)
        stag_v[sl] = stag_v[sl] + off
        return 0

    lax.fori_loop(0, PER_W // L, add_step, 0)

    g = [None] * NCHUNK
    w = [None] * NCHUNK

    def retire(j):
        g[j].wait()
        w[j] = pltpu.async_copy(
            bufs_v.at[j % NBUF],
            out_hbm.at[ob, pl.ds(or0 + j * CHUNK, CHUNK)],
            wsems[j % NBUF],
        )

    for j in range(NCHUNK):
        bi = j % NBUF
        if j >= NBUF:
            w[j - NBUF].wait()
        g[j] = pltpu.async_copy(
            table_hbm.at[stag_v.at[pl.ds(j * CHUNK, CHUNK)]],
            bufs_v.at[bi],
            gsems[bi],
        )
        if j >= DEPTH:
            retire(j - DEPTH)
    for j in range(NCHUNK - DEPTH, NCHUNK):
        retire(j)
    for j in range(NCHUNK - NBUF, NCHUNK):
        w[j].wait()


@jax.jit
def kernel(input_ids, table):
    ids1d = input_ids.reshape(N)
    off16 = jnp.asarray(OFF16, dtype=jnp.int32)
    mesh = plsc.VectorSubcoreMesh(core_axis_name="c", subcore_axis_name="s")
    out = pl.kernel(
        _body,
        mesh=mesh,
        out_type=jax.ShapeDtypeStruct((B, S * H, D), jnp.float32),
        compiler_params=pltpu.CompilerParams(use_tc_tiling_on_sc=False),
        scratch_types=[
            pltpu.VMEM((PER_W,), jnp.int32),
            pltpu.VMEM((NBUF, CHUNK, D), jnp.float32),
            pltpu.VMEM((L,), jnp.int32),
        ]
        + [pltpu.SemaphoreType.DMA] * (2 * NBUF),
    )(ids1d, table, off16)
    return out.reshape(B, S, H, D)
